# Initial kernel scaffold; baseline (speedup 1.0000x reference)
import jax
import jax.numpy as jnp
from jax.experimental import pallas as pl


def kernel(f0_indices, f1_indices, f2_indices, f3_indices, table_0, table_1, table_2, table_3, pw_0, pw_1, pw_2, pw_3):
    raise NotImplementedError("write your pallas kernel here")


# speedup vs baseline: 1.2271x; 1.2271x over previous
import functools

import jax
import jax.numpy as jnp
from jax import lax
from jax.experimental import pallas as pl
from jax.experimental.pallas import tpu as pltpu
from jax.experimental.pallas import tpu_sc as plsc

B = 4096
DIM = 64
LENS = (10, 10, 12, 12)
NW = 32
CH = B // NW
LMAX = max(LENS)
NVREG = DIM // 16


def _sc_body(i0, i1, i2, i3, t0, t1, t2, t3, p0, p1, p2, p3,
             o0, o1, o2, o3, lossp,
             idx_v, rows_v, out_v, pw_v, lsum_v, sem):
    wid = lax.axis_index("s") * 2 + lax.axis_index("c")
    lsum = jnp.zeros((16,), jnp.float32)

    feats = ((i0, t0, p0, o0, LENS[0]),
             (i1, t1, p1, o1, LENS[1]),
             (i2, t2, p2, o2, LENS[2]),
             (i3, t3, p3, o3, LENS[3]))

    for idx_h, tab_h, pw_h, out_h, L in feats:
        pltpu.sync_copy(idx_h.at[wid], idx_v.at[pl.ds(0, L)])
        pltpu.sync_copy(pw_h, pw_v.at[pl.ds(0, L)])

        copies = [
            pltpu.async_copy(tab_h.at[idx_v.at[j]],
                             rows_v.at[pl.ds(j * 128, 128)], sem)
            for j in range(L)
        ]
        for c in copies:
            c.wait()

        def bag_body(b, ls, L=L):
            base = b * L
            accs = [jnp.zeros((16,), jnp.float32) for _ in range(NVREG)]
            for l in range(L):
                pwl = pw_v[l]
                for d in range(NVREG):
                    accs[d] = accs[d] + pwl * rows_v[base + l, pl.ds(d * 16, 16)]
            for d in range(NVREG):
                out_v[b, pl.ds(d * 16, 16)] = accs[d]
            return ls + accs[0] + accs[1] + accs[2] + accs[3]

        lsum = lax.fori_loop(0, CH, bag_body, lsum)
        pltpu.sync_copy(out_v, out_h.at[pl.ds(wid * CH, CH)])

    lsum_v[...] = lsum
    pltpu.sync_copy(lsum_v, lossp.at[wid])


@jax.jit
def _run(idxs, tabs, pws):
    f32 = jnp.float32
    out_type = [jax.ShapeDtypeStruct((B, DIM), f32) for _ in range(4)]
    out_type.append(jax.ShapeDtypeStruct((NW, 16), f32))
    k = functools.partial(
        pl.kernel,
        out_type=out_type,
        mesh=plsc.VectorSubcoreMesh(core_axis_name="c", subcore_axis_name="s"),
        scratch_types=[
            pltpu.VMEM((LMAX, 128), jnp.int32),
            pltpu.VMEM((CH * LMAX, DIM), f32),
            pltpu.VMEM((CH, DIM), f32),
            pltpu.VMEM((LMAX, 16), f32),
            pltpu.VMEM((16,), f32),
            pltpu.SemaphoreType.DMA,
        ],
        compiler_params=pltpu.CompilerParams(use_tc_tiling_on_sc=False),
    )(_sc_body)
    return k(*idxs, *tabs, *pws)


def kernel(f0_indices, f1_indices, f2_indices, f3_indices,
           table_0, table_1, table_2, table_3,
           pw_0, pw_1, pw_2, pw_3):
    idxs = []
    for f, L in zip((f0_indices, f1_indices, f2_indices, f3_indices), LENS):
        idxs.append(f.astype(jnp.int32).reshape(NW, L, 128))
    pws = [jnp.broadcast_to(pw.astype(jnp.float32)[:, None], (L, 16))
           for pw, L in zip((pw_0, pw_1, pw_2, pw_3), LENS)]
    o0, o1, o2, o3, lossp = _run(tuple(idxs),
                                 (table_0, table_1, table_2, table_3),
                                 tuple(pws))
    pred = jnp.concatenate([o0, o1, o2, o3], axis=1)
    loss = jnp.sum(lossp) / (B * 4 * DIM)
    return (loss, pred)
